# baseline (speedup 1.0000x reference)
import functools

import jax
import jax.numpy as jnp
from jax import lax
from jax.experimental import pallas as pl
from jax.experimental.pallas import tpu as pltpu
from jax.experimental.pallas import tpu_sc as plsc

NUM_FRAMES = 16
T = 512
C, H, W = 3, 224, 224
R = C * H * W
NC, NS = 2, 16
NW = NC * NS
PAIRS_PW = (C * H) // NW
CHUNKS_PW = PAIRS_PW * 2
PR = (W // 8) * (T // 128) * 8
HPR = PR // 2
OW = W // 2


def _sc_gather(flat, idx32):
    mesh = plsc.VectorSubcoreMesh(core_axis_name="c", subcore_axis_name="s")

    @functools.partial(
        pl.kernel,
        out_type=jax.ShapeDtypeStruct((R * NUM_FRAMES,), jnp.float32),
        mesh=mesh,
        scratch_types=[
            pltpu.VMEM((32,), jnp.int32),
            pltpu.VMEM((HPR, 128), jnp.float32),
            pltpu.VMEM((HPR, 128), jnp.float32),
            pltpu.VMEM((OW * NUM_FRAMES,), jnp.float32),
            pltpu.VMEM((OW * NUM_FRAMES,), jnp.float32),
            pltpu.SemaphoreType.DMA,
            pltpu.SemaphoreType.DMA,
            pltpu.SemaphoreType.DMA,
            pltpu.SemaphoreType.DMA,
        ],
        compiler_params=pltpu.CompilerParams(needs_layout_passes=False),
    )
    def k(flat_hbm, idx_hbm, out_hbm, idx_v, bufA, bufB, oA, oB,
          giA, giB, goA, goB):
        wid = lax.axis_index("s") * NC + lax.axis_index("c")
        bufs, obufs = (bufA, bufB), (oA, oB)
        isems, osems = (giA, giB), (goA, goB)
        pltpu.sync_copy(idx_hbm, idx_v)
        ft8 = idx_v[pl.ds(0, 16)]
        fl = idx_v[pl.ds(16, 16)]
        base = wid * PAIRS_PW

        def in_copy(ch, p):
            pair = base + ch // 2
            row0 = pair * PR + (ch % 2) * HPR
            return pltpu.make_async_copy(
                flat_hbm.at[pl.ds(row0, HPR)], bufs[p], isems[p])

        def out_copy(ch, p):
            pair = base + ch // 2
            o0 = (pair * W + (ch % 2) * OW) * NUM_FRAMES
            return pltpu.make_async_copy(
                obufs[p], out_hbm.at[pl.ds(o0, OW * NUM_FRAMES)], osems[p])

        in_copy(0, 0).start()
        for ch in range(CHUNKS_PW):
            p = ch & 1
            if ch + 1 < CHUNKS_PW:
                in_copy(ch + 1, 1 - p).start()
            in_copy(ch, p).wait()
            if ch >= 2:
                out_copy(ch - 2, p).wait()

            def body(wt, _, p=p):
                rbase = wt * 32
                for ws in range(8):
                    rows = ft8 + (rbase + ws)
                    vals = plsc.load_gather(bufs[p], [rows, fl])
                    obufs[p][pl.ds((wt * 8 + ws) * NUM_FRAMES, 16)] = vals
                return 0

            lax.fori_loop(0, W // 16, body, 0)
            out_copy(ch, p).start()
        for ch in (CHUNKS_PW - 2, CHUNKS_PW - 1):
            out_copy(ch, ch & 1).wait()

    return k(flat, idx32)


def kernel(vid):
    tix = jax.random.randint(jax.random.key(42), (NUM_FRAMES,), 0, vid.shape[0])
    tix = tix.astype(jnp.int32)
    idx32 = jnp.concatenate([(tix >> 7) * 8, tix & 127])
    flat = (jnp.transpose(vid, (1, 2, 3, 0))
            .reshape(C, H, W // 8, 8, T // 128, 128)
            .transpose(0, 1, 2, 4, 3, 5)
            .reshape(R * (T // 128), 128))
    out = _sc_gather(flat, idx32)
    return jnp.transpose(out.reshape(C, H, W, NUM_FRAMES), (3, 0, 1, 2))


# speedup vs baseline: 1.1233x; 1.1233x over previous
import functools

import jax
import jax.numpy as jnp
from jax import lax
from jax.experimental import pallas as pl
from jax.experimental.pallas import tpu as pltpu
from jax.experimental.pallas import tpu_sc as plsc

NUM_FRAMES = 16
T = 512
C, H, W = 3, 224, 224
R = C * H * W
NC, NS = 2, 16
NW = NC * NS
PAIRS_PW = (C * H) // NW
NSPLIT = 4
CHUNKS_PW = PAIRS_PW * NSPLIT
PR = (W // 8) * (T // 128) * 8
CPR = PR // NSPLIT
OW = W // NSPLIT
NBUF = 4


def _sc_gather(flat, idx32):
    mesh = plsc.VectorSubcoreMesh(core_axis_name="c", subcore_axis_name="s")

    @functools.partial(
        pl.kernel,
        out_type=jax.ShapeDtypeStruct((R * NUM_FRAMES,), jnp.float32),
        mesh=mesh,
        scratch_types=[
            pltpu.VMEM((32,), jnp.int32),
        ] + [pltpu.VMEM((CPR, 128), jnp.float32)] * NBUF
          + [pltpu.VMEM((OW * NUM_FRAMES,), jnp.float32)] * 2
          + [pltpu.SemaphoreType.DMA] * (NBUF + 2),
        compiler_params=pltpu.CompilerParams(needs_layout_passes=False),
    )
    def k(flat_hbm, idx_hbm, out_hbm, idx_v, b0, b1, b2, b3, oA, oB,
          gi0, gi1, gi2, gi3, goA, goB):
        wid = lax.axis_index("s") * NC + lax.axis_index("c")
        bufs, obufs = (b0, b1, b2, b3), (oA, oB)
        isems, osems = (gi0, gi1, gi2, gi3), (goA, goB)
        pltpu.sync_copy(idx_hbm, idx_v)
        ft8 = idx_v[pl.ds(0, 16)]
        fl = idx_v[pl.ds(16, 16)]
        row_base = wid * PAIRS_PW * PR
        out_base = wid * PAIRS_PW * W * NUM_FRAMES

        def in_copy(ch, p):
            return pltpu.make_async_copy(
                flat_hbm.at[pl.ds(row_base + ch * CPR, CPR)],
                bufs[p], isems[p])

        def out_copy(ch, p):
            return pltpu.make_async_copy(
                obufs[p],
                out_hbm.at[pl.ds(out_base + ch * OW * NUM_FRAMES,
                                 OW * NUM_FRAMES)],
                osems[p])

        for ch in range(NBUF - 1):
            in_copy(ch, ch).start()

        def group(g, _):
            for b in range(NBUF):
                ch = g * NBUF + b
                nxt = ch + NBUF - 1

                @pl.when(nxt < CHUNKS_PW)
                def _():
                    in_copy(nxt, (b + NBUF - 1) % NBUF).start()

                in_copy(ch, b).wait()

                @pl.when(ch >= 2)
                def _():
                    out_copy(ch - 2, b & 1).wait()

                def body(wt, _, b=b, ch=ch):
                    rbase = wt * 32
                    for ws in range(8):
                        rows = ft8 + (rbase + ws)
                        vals = plsc.load_gather(bufs[b], [rows, fl])
                        obufs[b & 1][pl.ds((wt * 8 + ws) * NUM_FRAMES, 16)] \
                            = vals
                    return 0

                lax.fori_loop(0, OW // 8, body, 0)
                out_copy(ch, b & 1).start()
            return 0

        lax.fori_loop(0, CHUNKS_PW // NBUF, group, 0)
        for ch in (CHUNKS_PW - 2, CHUNKS_PW - 1):
            out_copy(ch, ch & 1).wait()

    return k(flat, idx32)


def kernel(vid):
    tix = jax.random.randint(jax.random.key(42), (NUM_FRAMES,), 0, vid.shape[0])
    tix = tix.astype(jnp.int32)
    idx32 = jnp.concatenate([(tix >> 7) * 8, tix & 127])
    flat = (jnp.transpose(vid, (1, 2, 3, 0))
            .reshape(C, H, W // 8, 8, T // 128, 128)
            .transpose(0, 1, 2, 4, 3, 5)
            .reshape(R * (T // 128), 128))
    out = _sc_gather(flat, idx32)
    return jnp.transpose(out.reshape(C, H, W, NUM_FRAMES), (3, 0, 1, 2))


# speedup vs baseline: 1.1379x; 1.0130x over previous
import functools

import numpy as np

import jax
import jax.numpy as jnp
from jax import lax
from jax.experimental import pallas as pl
from jax.experimental.pallas import tpu as pltpu
from jax.experimental.pallas import tpu_sc as plsc

NUM_FRAMES = 16
T = 512
C, H, W = 3, 224, 224
R = C * H * W
NC, NS = 2, 16
NW = NC * NS
PAIRS_PW = (C * H) // NW
NSPLIT = 4
CHUNKS_PW = PAIRS_PW * NSPLIT
PR = (W // 8) * (T // 128) * 8
CPR = PR // NSPLIT
OW = W // NSPLIT
NBUF = 4


def _sc_gather(flat, idx32):
    mesh = plsc.VectorSubcoreMesh(core_axis_name="c", subcore_axis_name="s")

    @functools.partial(
        pl.kernel,
        out_type=jax.ShapeDtypeStruct((R * NUM_FRAMES,), jnp.float32),
        mesh=mesh,
        scratch_types=[
            pltpu.VMEM((32,), jnp.int32),
        ] + [pltpu.VMEM((CPR, 128), jnp.float32)] * NBUF
          + [pltpu.VMEM((OW * NUM_FRAMES,), jnp.float32)] * 2
          + [pltpu.SemaphoreType.DMA] * (NBUF + 2),
        compiler_params=pltpu.CompilerParams(needs_layout_passes=False),
    )
    def k(flat_hbm, idx_hbm, out_hbm, idx_v, b0, b1, b2, b3, oA, oB,
          gi0, gi1, gi2, gi3, goA, goB):
        wid = lax.axis_index("s") * NC + lax.axis_index("c")
        bufs, obufs = (b0, b1, b2, b3), (oA, oB)
        isems, osems = (gi0, gi1, gi2, gi3), (goA, goB)
        pltpu.sync_copy(idx_hbm, idx_v)
        ft8 = idx_v[pl.ds(0, 16)]
        fl = idx_v[pl.ds(16, 16)]
        row_base = wid * PAIRS_PW * PR
        out_base = wid * PAIRS_PW * W * NUM_FRAMES

        def in_copy(ch, p):
            return pltpu.make_async_copy(
                flat_hbm.at[pl.ds(row_base + ch * CPR, CPR)],
                bufs[p], isems[p])

        def out_copy(ch, p):
            return pltpu.make_async_copy(
                obufs[p],
                out_hbm.at[pl.ds(out_base + ch * OW * NUM_FRAMES,
                                 OW * NUM_FRAMES)],
                osems[p])

        for ch in range(NBUF - 1):
            in_copy(ch, ch).start()

        def group(g, _):
            for b in range(NBUF):
                ch = g * NBUF + b
                nxt = ch + NBUF - 1

                @pl.when(nxt < CHUNKS_PW)
                def _():
                    in_copy(nxt, (b + NBUF - 1) % NBUF).start()

                in_copy(ch, b).wait()

                @pl.when(ch >= 2)
                def _():
                    out_copy(ch - 2, b & 1).wait()

                def body(wt, _, b=b, ch=ch):
                    rbase = wt * 32
                    for ws in range(8):
                        rows = ft8 + (rbase + ws)
                        vals = plsc.load_gather(bufs[b], [rows, fl])
                        obufs[b & 1][pl.ds((wt * 8 + ws) * NUM_FRAMES, 16)] \
                            = vals
                    return 0

                lax.fori_loop(0, OW // 8, body, 0)
                out_copy(ch, b & 1).start()
            return 0

        lax.fori_loop(0, CHUNKS_PW // NBUF, group, 0)
        for ch in (CHUNKS_PW - 2, CHUNKS_PW - 1):
            out_copy(ch, ch & 1).wait()

    return k(flat, idx32)


_TIX = np.asarray(
    jax.random.randint(jax.random.key(42), (NUM_FRAMES,), 0, T)
).astype(np.int32)
_IDX32 = np.concatenate([(_TIX >> 7) * 8, _TIX & 127])


def kernel(vid):
    idx32 = jnp.asarray(_IDX32)
    flat = (jnp.transpose(vid, (1, 2, 3, 0))
            .reshape(C, H, W // 8, 8, T // 128, 128)
            .transpose(0, 1, 2, 4, 3, 5)
            .reshape(R * (T // 128), 128))
    out = _sc_gather(flat, idx32)
    return jnp.transpose(out.reshape(C, H, W, NUM_FRAMES), (3, 0, 1, 2))


# speedup vs baseline: 1.4662x; 1.2885x over previous
import functools

import numpy as np

import jax
import jax.numpy as jnp
from jax import lax
from jax.experimental import pallas as pl
from jax.experimental.pallas import tpu as pltpu
from jax.experimental.pallas import tpu_sc as plsc

NUM_FRAMES = 16
T = 512
C, H, W = 3, 224, 224
R = C * H * W
NC, NS = 2, 16
NW = NC * NS
PAIRS_PW = (C * H) // NW
PR = (W // 8) * (T // 128) * 8
NQ = 4
QROWS = (256, 256, 256, 128)
QWT2 = (0, 0, 1, 1)
QL0 = (0, 64, 0, 64)
QNL = (64, 64, 64, 32)


def _sc_gather(flat, idx32):
    mesh = plsc.VectorSubcoreMesh(core_axis_name="c", subcore_axis_name="s")

    @functools.partial(
        pl.kernel,
        out_type=jax.ShapeDtypeStruct((2, C * 28 * 16, 8, 128), jnp.float32),
        mesh=mesh,
        scratch_types=[
            pltpu.VMEM((32,), jnp.int32),
            pltpu.VMEM((256, 128), jnp.float32),
            pltpu.VMEM((256, 128), jnp.float32),
            pltpu.VMEM((256, 128), jnp.float32),
            pltpu.VMEM((128, 128), jnp.float32),
            pltpu.VMEM((2, 1, 8, 128), jnp.float32),
            pltpu.VMEM((2, 1, 8, 128), jnp.float32),
        ] + [pltpu.SemaphoreType.DMA] * 6,
        compiler_params=pltpu.CompilerParams(needs_layout_passes=False),
    )
    def k(flat_hbm, idx_hbm, out_hbm, idx_v, b0, b1, b2, b3, oA, oB,
          gi0, gi1, gi2, gi3, goA, goB):
        wid = lax.axis_index("s") * NC + lax.axis_index("c")
        bufs, obufs = (b0, b1, b2, b3), (oA, oB)
        isems, osems = (gi0, gi1, gi2, gi3), (goA, goB)
        pltpu.sync_copy(idx_hbm, idx_v)
        ft8 = idx_v[pl.ds(0, 16)]
        fl = idx_v[pl.ds(16, 16)]
        jlane = lax.iota(jnp.int32, 16)
        jrt = jlane >> 3
        jzero = jnp.zeros((16,), jnp.int32)
        jsub = jlane & 7
        base = wid * PAIRS_PW

        def in_copy(p, q):
            row0 = (base + p) * PR + q * 256
            return pltpu.make_async_copy(
                flat_hbm.at[pl.ds(row0, QROWS[q])], bufs[q], isems[q])

        def out_copy(p, half):
            gp = base + p
            c = gp // H
            hh = gp % H
            cht = c * 28 + hh // 8
            ct = cht * 16 + half * 8 + (hh % 8)
            return pltpu.make_async_copy(
                obufs[half],
                out_hbm.at[:, pl.ds(ct, 1)],
                osems[half])

        for q in range(NQ - 1):
            in_copy(0, q).start()

        def pair_step(p, _):
            for q in range(NQ):
                if q == 0:
                    in_copy(p, 3).start()
                else:
                    @pl.when(p < PAIRS_PW - 1)
                    def _():
                        in_copy(p + 1, q - 1).start()

                in_copy(p, q).wait()

                if q in (0, 2):
                    @pl.when(p >= 1)
                    def _():
                        out_copy(p - 1, q // 2).wait()

                def body(wt, _, q=q):
                    rbase = wt * 32
                    for ws in range(8):
                        rows = ft8 + (rbase + ws)
                        vals = plsc.load_gather(bufs[q], [rows, fl])
                        wloc = jnp.full((16,), QL0[q] + wt * 8 + ws,
                                        jnp.int32)
                        plsc.store_scatter(
                            obufs[q // 2], [jrt, jzero, jsub, wloc], vals)
                    return 0

                lax.fori_loop(0, QNL[q] // 8, body, 0)
                if q in (1, 3):
                    out_copy(p, q // 2).start()
            return 0

        lax.fori_loop(0, PAIRS_PW, pair_step, 0)
        out_copy(PAIRS_PW - 1, 0).wait()
        out_copy(PAIRS_PW - 1, 1).wait()

    return k(flat, idx32)


_TIX = np.asarray(
    jax.random.randint(jax.random.key(42), (NUM_FRAMES,), 0, T)
).astype(np.int32)
_IDX32 = np.concatenate([(_TIX >> 7) * 8, _TIX & 127])


def kernel(vid):
    idx32 = jnp.asarray(_IDX32)
    flat = (jnp.transpose(vid, (1, 2, 3, 0))
            .reshape(C, H, W // 8, 8, T // 128, 128)
            .transpose(0, 1, 2, 4, 3, 5)
            .reshape(R * (T // 128), 128))
    out = _sc_gather(flat, idx32)
    out = (out.reshape(2, C * 28, 2, 8, 8, 128)
           .transpose(0, 4, 1, 3, 2, 5)
           .reshape(NUM_FRAMES, C, 28, 8, 2, 128)
           .reshape(NUM_FRAMES, C, H, 256)[..., :W])
    return out


# speedup vs baseline: 1.4671x; 1.0006x over previous
import functools

import numpy as np

import jax
import jax.numpy as jnp
from jax import lax
from jax.experimental import pallas as pl
from jax.experimental.pallas import tpu as pltpu
from jax.experimental.pallas import tpu_sc as plsc

NUM_FRAMES = 16
T = 512
C, H, W = 3, 224, 224
R = C * H * W
NC, NS = 2, 16
NW = NC * NS
PAIRS_PW = (C * H) // NW
PR = (W // 8) * (T // 128) * 8
NQ = 4
QROWS = (256, 256, 256, 128)
QL0 = (0, 64, 0, 64)
QNL = (64, 64, 64, 32)


def _sc_gather(flat, idx32):
    mesh = plsc.VectorSubcoreMesh(core_axis_name="c", subcore_axis_name="s")

    @functools.partial(
        pl.kernel,
        out_type=jax.ShapeDtypeStruct((2, C * 28 * 16, 8, 128), jnp.float32),
        mesh=mesh,
        scratch_types=[
            pltpu.VMEM((32,), jnp.int32),
            pltpu.VMEM((256, 128), jnp.float32),
            pltpu.VMEM((256, 128), jnp.float32),
            pltpu.VMEM((256, 128), jnp.float32),
            pltpu.VMEM((128, 128), jnp.float32),
            pltpu.VMEM((2, 1, 8, 128), jnp.float32),
            pltpu.VMEM((2, 1, 8, 128), jnp.float32),
        ] + [pltpu.SemaphoreType.DMA] * 6,
        compiler_params=pltpu.CompilerParams(needs_layout_passes=False),
    )
    def k(flat_hbm, idx_hbm, out_hbm, idx_v, b0, b1, b2, b3, oA, oB,
          gi0, gi1, gi2, gi3, goA, goB):
        wid = lax.axis_index("s") * NC + lax.axis_index("c")
        bufs, obufs = (b0, b1, b2, b3), (oA, oB)
        isems, osems = (gi0, gi1, gi2, gi3), (goA, goB)
        pltpu.sync_copy(idx_hbm, idx_v)
        ft8 = idx_v[pl.ds(0, 16)]
        fl = idx_v[pl.ds(16, 16)]
        jlane = lax.iota(jnp.int32, 16)
        jrt = jlane >> 3
        jzero = jnp.zeros((16,), jnp.int32)
        jsub = jlane & 7
        base = wid * PAIRS_PW

        def in_copy(p, q):
            row0 = (base + p) * PR + q * 256
            return pltpu.make_async_copy(
                flat_hbm.at[pl.ds(row0, QROWS[q])], bufs[q], isems[q])

        def out_copy(p, half):
            gp = base + p
            c = gp // H
            hh = gp % H
            cht = c * 28 + hh // 8
            ct = cht * 16 + half * 8 + (hh % 8)
            return pltpu.make_async_copy(
                obufs[half],
                out_hbm.at[:, pl.ds(ct, 1)],
                osems[half])

        for q in range(NQ - 1):
            in_copy(0, q).start()

        def pair_step(p, _):
            for q in range(NQ):
                if q == 0:
                    in_copy(p, 3).start()
                else:
                    @pl.when(p < PAIRS_PW - 1)
                    def _():
                        in_copy(p + 1, q - 1).start()

                in_copy(p, q).wait()

                if q in (0, 2):
                    @pl.when(p >= 1)
                    def _():
                        out_copy(p - 1, q // 2).wait()

                def body(wt, _, q=q):
                    rbase = wt * 32
                    for ws in range(8):
                        rows = ft8 + (rbase + ws)
                        vals = plsc.load_gather(bufs[q], [rows, fl])
                        wloc = jnp.full((16,), QL0[q] + wt * 8 + ws,
                                        jnp.int32)
                        plsc.store_scatter(
                            obufs[q // 2], [jrt, jzero, jsub, wloc], vals)
                    return 0

                lax.fori_loop(0, QNL[q] // 8, body, 0)
                if q in (1, 3):
                    out_copy(p, q // 2).start()
            return 0

        lax.fori_loop(0, PAIRS_PW, pair_step, 0)
        out_copy(PAIRS_PW - 1, 0).wait()
        out_copy(PAIRS_PW - 1, 1).wait()

    return k(flat, idx32)


_TIX = np.asarray(
    jax.random.randint(jax.random.key(42), (NUM_FRAMES,), 0, T)
).astype(np.int32)
_IDX32 = np.concatenate([(_TIX >> 7) * 8, _TIX & 127])


def kernel(vid):
    idx32 = jnp.asarray(_IDX32)
    flat = (jnp.transpose(vid, (1, 2, 3, 0))
            .reshape(C, H, W // 8, 8, T // 128, 128)
            .transpose(0, 1, 2, 4, 3, 5)
            .reshape(R * (T // 128), 128))
    out = _sc_gather(flat, idx32)
    out = (out.reshape(2, C * 28, 2, 8, 8, 128)
           .transpose(0, 4, 1, 3, 2, 5)
           .reshape(NUM_FRAMES, C, 28, 8, 2, 128)
           .reshape(NUM_FRAMES, C, H, 256)[..., :W])
    return out
